# baseline (speedup 1.0000x reference)
import functools

import jax
import jax.numpy as jnp
from jax import lax
from jax.experimental import pallas as pl
from jax.experimental.pallas import tpu as pltpu
from jax.experimental.pallas import tpu_sc as plsc

_NODE_FEATS = 128
_STAT_MEDIAN = 0.0
_STAT_SCALE = 2.0 / (1.0 - (-1.0))
_N_NODES = 10000
_N_EDGES = 320000

_NW = 32
_E_PER_W = _N_EDGES // _NW
_CHUNK = 80
_CH_PER_W = _E_PER_W // _CHUNK
_IDX_BLK = 25
_N_BLKS = _CH_PER_W // _IDX_BLK

_mesh = plsc.VectorSubcoreMesh(core_axis_name="c", subcore_axis_name="s")


@functools.partial(
    pl.kernel,
    mesh=_mesh,
    out_type=jax.ShapeDtypeStruct((_N_EDGES, _NODE_FEATS), jnp.float32),
    scratch_types=[
        pltpu.VMEM_SHARED((_N_NODES, _NODE_FEATS), jnp.float32),
        pltpu.VMEM((2 * _IDX_BLK * _CHUNK,), jnp.int32),
        pltpu.VMEM((2 * _IDX_BLK * _CHUNK,), jnp.int32),
        pltpu.VMEM((2, _CHUNK, _NODE_FEATS), jnp.float32),
        pltpu.SemaphoreType.DMA((2,)),
        pltpu.SemaphoreType.DMA((2,)),
        pltpu.SemaphoreType.DMA((2,)),
        pltpu.SemaphoreType.DMA((2,)),
    ],
)
def _edge_kernel(norm_hbm, neg_hbm, src_hbm, dst_hbm, out_hbm,
                 table, sidx, didx, rows, sem_s, sem_d, sem_o, sem_i):
    wid = lax.axis_index("s") * 2 + lax.axis_index("c")
    ebase = wid * _E_PER_W
    sid = lax.axis_index("s")

    rows_per_sub = 624
    tslice = pl.ds(sid * rows_per_sub, rows_per_sub)
    pltpu.async_copy(neg_hbm.at[tslice], table.at[tslice], sem_o.at[0])
    rem = pl.ds(16 * rows_per_sub, _N_NODES - 16 * rows_per_sub)

    @pl.when(sid == 0)
    def _():
        pltpu.async_copy(neg_hbm.at[rem], table.at[rem], sem_o.at[1])

    _BLK_E = _IDX_BLK * _CHUNK

    def fetch_idx(j, jbuf):
        ibase = ebase + j * _BLK_E
        vsl = pl.ds(jbuf * _BLK_E, _BLK_E)
        pltpu.async_copy(src_hbm.at[pl.ds(ibase, _BLK_E)], sidx.at[vsl],
                         sem_i.at[jbuf])
        pltpu.async_copy(dst_hbm.at[pl.ds(ibase, _BLK_E)], didx.at[vsl],
                         sem_i.at[jbuf])

    def wait_idx(j, jbuf):
        ibase = ebase + j * _BLK_E
        vsl = pl.ds(jbuf * _BLK_E, _BLK_E)
        pltpu.make_async_copy(src_hbm.at[pl.ds(ibase, _BLK_E)],
                              sidx.at[vsl], sem_i.at[jbuf]).wait()
        pltpu.make_async_copy(dst_hbm.at[pl.ds(ibase, _BLK_E)],
                              didx.at[vsl], sem_i.at[jbuf]).wait()

    fetch_idx(0, 0)
    fetch_idx(1, 1)
    pltpu.make_async_copy(neg_hbm.at[tslice], table.at[tslice],
                          sem_o.at[0]).wait()

    @pl.when(sid == 0)
    def _():
        pltpu.make_async_copy(neg_hbm.at[rem], table.at[rem],
                              sem_o.at[1]).wait()

    plsc.subcore_barrier()

    def idx_refs(i):
        off = ((i // _IDX_BLK) % 2) * _BLK_E + (i % _IDX_BLK) * _CHUNK
        return (sidx.at[pl.ds(off, _CHUNK)], didx.at[pl.ds(off, _CHUNK)])

    def issue_src(i, b):
        @pl.when(i % _IDX_BLK == 0)
        def _():
            wait_idx(i // _IDX_BLK, (i // _IDX_BLK) % 2)

        s_ix, _ = idx_refs(i)
        pltpu.async_copy(norm_hbm.at[s_ix], rows.at[b], sem_s.at[b])

    def wait_src(i, b):
        s_ix, _ = idx_refs(i)
        pltpu.make_async_copy(norm_hbm.at[s_ix], rows.at[b],
                              sem_s.at[b]).wait()

    def issue_dst_add(i, b):
        _, d_ix = idx_refs(i)
        pltpu.async_copy(table.at[d_ix], rows.at[b], sem_d.at[b], add=True)

    def wait_dst_add(i, b):
        _, d_ix = idx_refs(i)
        pltpu.make_async_copy(table.at[d_ix], rows.at[b],
                              sem_d.at[b]).wait()

    def prefetch_idx(i):
        j2 = i // _IDX_BLK + 2

        @pl.when((i % _IDX_BLK == _IDX_BLK - 1) & (j2 < _N_BLKS))
        def _():
            fetch_idx(j2, j2 % 2)

    def out_slice(i):
        return out_hbm.at[pl.ds(ebase + i * _CHUNK, _CHUNK)]

    def wait_out(i, b):
        pltpu.make_async_copy(rows.at[b], out_slice(i), sem_o.at[b]).wait()

    wait_idx(0, 0)
    s_ix0, _ = idx_refs(0)
    pltpu.async_copy(norm_hbm.at[s_ix0], rows.at[0], sem_s.at[0])

    def body(i0, carry):
        for b2 in range(2):
            i = i0 * 2 + b2
            bnext = 1 - b2
            wait_src(i, b2)
            issue_dst_add(i, b2)
            if b2 == 0:
                @pl.when(i0 > 0)
                def _():
                    wait_out(i - 1, bnext)
            else:
                wait_out(i - 1, bnext)
            issue_src(i + 1, bnext)
            wait_dst_add(i, b2)
            prefetch_idx(i)
            pltpu.async_copy(rows.at[b2], out_slice(i), sem_o.at[b2])
        return carry

    lax.fori_loop(0, (_CH_PER_W - 1) // 2, body, 0)

    wait_src(_CH_PER_W - 1, 0)
    issue_dst_add(_CH_PER_W - 1, 0)
    wait_out(_CH_PER_W - 2, 1)
    wait_dst_add(_CH_PER_W - 1, 0)
    pltpu.async_copy(rows.at[0], out_slice(_CH_PER_W - 1), sem_o.at[0])
    wait_out(_CH_PER_W - 1, 0)


def _norm_body(x_ref, o_ref, n_ref):
    norm = (x_ref[...] - _STAT_MEDIAN) * _STAT_SCALE
    o_ref[...] = norm
    n_ref[...] = -norm


_norm_call = pl.pallas_call(
    _norm_body,
    out_shape=(
        jax.ShapeDtypeStruct((_N_NODES, _NODE_FEATS), jnp.float32),
        jax.ShapeDtypeStruct((_N_NODES, _NODE_FEATS), jnp.float32),
    ),
)


def kernel(node_feature, edge_index):
    src = edge_index[0].astype(jnp.int32)
    dst = edge_index[1].astype(jnp.int32)
    norm, neg = _norm_call(node_feature)
    edge_feature = _edge_kernel(norm, neg, src, dst)
    return (norm, edge_feature)


# speedup vs baseline: 1.2921x; 1.2921x over previous
import functools

import jax
import jax.numpy as jnp
from jax import lax
from jax.experimental import pallas as pl
from jax.experimental.pallas import tpu as pltpu
from jax.experimental.pallas import tpu_sc as plsc

_NODE_FEATS = 128
_STAT_MEDIAN = 0.0
_STAT_SCALE = 2.0 / (1.0 - (-1.0))
_N_NODES = 10000
_N_EDGES = 320000

_NW = 32
_E_PER_W = _N_EDGES // _NW
_CHUNK = 80
_CH_PER_W = _E_PER_W // _CHUNK
_IDX_BLK = 25
_N_BLKS = _CH_PER_W // _IDX_BLK

_mesh = plsc.VectorSubcoreMesh(core_axis_name="c", subcore_axis_name="s")


@functools.partial(
    pl.kernel,
    mesh=_mesh,
    out_type=jax.ShapeDtypeStruct((_N_EDGES, _NODE_FEATS), jnp.float32),
    scratch_types=[
        pltpu.VMEM_SHARED((_N_NODES, _NODE_FEATS), jnp.float32),
        pltpu.VMEM((2 * _IDX_BLK * _CHUNK,), jnp.int32),
        pltpu.VMEM((2 * _IDX_BLK * _CHUNK,), jnp.int32),
        pltpu.VMEM((4, _CHUNK, _NODE_FEATS), jnp.float32),
        pltpu.SemaphoreType.DMA((4,)),
        pltpu.SemaphoreType.DMA((4,)),
        pltpu.SemaphoreType.DMA((4,)),
        pltpu.SemaphoreType.DMA((2,)),
    ],
)
def _edge_kernel(norm_hbm, neg_hbm, src_hbm, dst_hbm, out_hbm,
                 table, sidx, didx, rows, sem_s, sem_d, sem_o, sem_i):
    wid = lax.axis_index("s") * 2 + lax.axis_index("c")
    ebase = wid * _E_PER_W
    sid = lax.axis_index("s")

    rows_per_sub = 624
    tslice = pl.ds(sid * rows_per_sub, rows_per_sub)
    pltpu.async_copy(neg_hbm.at[tslice], table.at[tslice], sem_o.at[0])
    rem = pl.ds(16 * rows_per_sub, _N_NODES - 16 * rows_per_sub)

    @pl.when(sid == 0)
    def _():
        pltpu.async_copy(neg_hbm.at[rem], table.at[rem], sem_o.at[1])

    _BLK_E = _IDX_BLK * _CHUNK

    def fetch_idx(j, jbuf):
        ibase = ebase + j * _BLK_E
        vsl = pl.ds(jbuf * _BLK_E, _BLK_E)
        pltpu.async_copy(src_hbm.at[pl.ds(ibase, _BLK_E)], sidx.at[vsl],
                         sem_i.at[jbuf])
        pltpu.async_copy(dst_hbm.at[pl.ds(ibase, _BLK_E)], didx.at[vsl],
                         sem_i.at[jbuf])

    def wait_idx(j, jbuf):
        ibase = ebase + j * _BLK_E
        vsl = pl.ds(jbuf * _BLK_E, _BLK_E)
        pltpu.make_async_copy(src_hbm.at[pl.ds(ibase, _BLK_E)],
                              sidx.at[vsl], sem_i.at[jbuf]).wait()
        pltpu.make_async_copy(dst_hbm.at[pl.ds(ibase, _BLK_E)],
                              didx.at[vsl], sem_i.at[jbuf]).wait()

    fetch_idx(0, 0)
    fetch_idx(1, 1)
    pltpu.make_async_copy(neg_hbm.at[tslice], table.at[tslice],
                          sem_o.at[0]).wait()

    @pl.when(sid == 0)
    def _():
        pltpu.make_async_copy(neg_hbm.at[rem], table.at[rem],
                              sem_o.at[1]).wait()

    plsc.subcore_barrier()

    def idx_refs(i):
        off = ((i // _IDX_BLK) % 2) * _BLK_E + (i % _IDX_BLK) * _CHUNK
        return (sidx.at[pl.ds(off, _CHUNK)], didx.at[pl.ds(off, _CHUNK)])

    def issue_src(i, b):
        @pl.when(i % _IDX_BLK == 0)
        def _():
            wait_idx(i // _IDX_BLK, (i // _IDX_BLK) % 2)

        s_ix, _ = idx_refs(i)
        pltpu.async_copy(norm_hbm.at[s_ix], rows.at[b], sem_s.at[b])

    def wait_src(i, b):
        s_ix, _ = idx_refs(i)
        pltpu.make_async_copy(norm_hbm.at[s_ix], rows.at[b],
                              sem_s.at[b]).wait()

    def issue_dst_add(i, b):
        _, d_ix = idx_refs(i)
        pltpu.async_copy(table.at[d_ix], rows.at[b], sem_d.at[b], add=True)

    def wait_dst_add(i, b):
        _, d_ix = idx_refs(i)
        pltpu.make_async_copy(table.at[d_ix], rows.at[b],
                              sem_d.at[b]).wait()

    def prefetch_idx(i):
        j2 = i // _IDX_BLK + 2

        @pl.when((i % _IDX_BLK == _IDX_BLK - 1) & (j2 < _N_BLKS))
        def _():
            fetch_idx(j2, j2 % 2)

    def out_slice(i):
        return out_hbm.at[pl.ds(ebase + i * _CHUNK, _CHUNK)]

    def wait_out(i, b):
        pltpu.make_async_copy(rows.at[b], out_slice(i), sem_o.at[b]).wait()

    wait_idx(0, 0)
    for j in range(3):
        s_ixj, _ = idx_refs(j)
        pltpu.async_copy(norm_hbm.at[s_ixj], rows.at[j], sem_s.at[j])

    def body(i0, carry):
        for b2 in range(4):
            i = i0 * 4 + b2
            bg = (b2 + 3) % 4
            if b2 == 0:
                @pl.when(i0 > 0)
                def _():
                    wait_out(i - 1, bg)
            else:
                wait_out(i - 1, bg)
            @pl.when(i + 3 < _CH_PER_W)
            def _():
                issue_src(i + 3, bg)

            wait_src(i, b2)
            issue_dst_add(i, b2)
            wait_dst_add(i, b2)
            prefetch_idx(i)
            pltpu.async_copy(rows.at[b2], out_slice(i), sem_o.at[b2])
        return carry

    lax.fori_loop(0, (_CH_PER_W - 1) // 4, body, 0)

    _L = _CH_PER_W - 1
    wait_out(_L - 1, 3)
    wait_src(_L, 0)
    issue_dst_add(_L, 0)
    wait_dst_add(_L, 0)
    pltpu.async_copy(rows.at[0], out_slice(_L), sem_o.at[0])
    wait_out(_L, 0)


def _norm_body(x_ref, o_ref, n_ref):
    norm = (x_ref[...] - _STAT_MEDIAN) * _STAT_SCALE
    o_ref[...] = norm
    n_ref[...] = -norm


_norm_call = pl.pallas_call(
    _norm_body,
    out_shape=(
        jax.ShapeDtypeStruct((_N_NODES, _NODE_FEATS), jnp.float32),
        jax.ShapeDtypeStruct((_N_NODES, _NODE_FEATS), jnp.float32),
    ),
)


def kernel(node_feature, edge_index):
    src = edge_index[0].astype(jnp.int32)
    dst = edge_index[1].astype(jnp.int32)
    norm, neg = _norm_call(node_feature)
    edge_feature = _edge_kernel(norm, neg, src, dst)
    return (norm, edge_feature)


# speedup vs baseline: 1.2958x; 1.0029x over previous
import functools

import jax
import jax.numpy as jnp
from jax import lax
from jax.experimental import pallas as pl
from jax.experimental.pallas import tpu as pltpu
from jax.experimental.pallas import tpu_sc as plsc

_NODE_FEATS = 128
_STAT_MEDIAN = 0.0
_STAT_SCALE = 2.0 / (1.0 - (-1.0))
_N_NODES = 10000
_N_EDGES = 320000

_NW = 32
_E_PER_W = _N_EDGES // _NW
_CHUNK = 80
_CH_PER_W = _E_PER_W // _CHUNK
_IDX_BLK = 25
_N_BLKS = _CH_PER_W // _IDX_BLK

_mesh = plsc.VectorSubcoreMesh(core_axis_name="c", subcore_axis_name="s")


@functools.partial(
    pl.kernel,
    mesh=_mesh,
    out_type=jax.ShapeDtypeStruct((_N_EDGES, _NODE_FEATS), jnp.float32),
    scratch_types=[
        pltpu.VMEM_SHARED((_N_NODES, _NODE_FEATS), jnp.float32),
        pltpu.VMEM((2 * _IDX_BLK * _CHUNK,), jnp.int32),
        pltpu.VMEM((2 * _IDX_BLK * _CHUNK,), jnp.int32),
        pltpu.VMEM((4, _CHUNK, _NODE_FEATS), jnp.float32),
        pltpu.SemaphoreType.DMA((4,)),
        pltpu.SemaphoreType.DMA((4,)),
        pltpu.SemaphoreType.DMA((4,)),
        pltpu.SemaphoreType.DMA((2,)),
    ],
)
def _edge_kernel(norm_hbm, neg_hbm, src_hbm, dst_hbm, out_hbm,
                 table, sidx, didx, rows, sem_s, sem_d, sem_o, sem_i):
    wid = lax.axis_index("s") * 2 + lax.axis_index("c")
    ebase = wid * _E_PER_W
    sid = lax.axis_index("s")

    rows_per_sub = 624
    tslice = pl.ds(sid * rows_per_sub, rows_per_sub)
    pltpu.async_copy(neg_hbm.at[tslice], table.at[tslice], sem_o.at[0])
    rem = pl.ds(16 * rows_per_sub, _N_NODES - 16 * rows_per_sub)

    @pl.when(sid == 0)
    def _():
        pltpu.async_copy(neg_hbm.at[rem], table.at[rem], sem_o.at[1])

    _BLK_E = _IDX_BLK * _CHUNK

    def fetch_idx(j, jbuf):
        ibase = ebase + j * _BLK_E
        vsl = pl.ds(jbuf * _BLK_E, _BLK_E)
        pltpu.async_copy(src_hbm.at[pl.ds(ibase, _BLK_E)], sidx.at[vsl],
                         sem_i.at[jbuf])
        pltpu.async_copy(dst_hbm.at[pl.ds(ibase, _BLK_E)], didx.at[vsl],
                         sem_i.at[jbuf])

    def wait_idx(j, jbuf):
        ibase = ebase + j * _BLK_E
        vsl = pl.ds(jbuf * _BLK_E, _BLK_E)
        pltpu.make_async_copy(src_hbm.at[pl.ds(ibase, _BLK_E)],
                              sidx.at[vsl], sem_i.at[jbuf]).wait()
        pltpu.make_async_copy(dst_hbm.at[pl.ds(ibase, _BLK_E)],
                              didx.at[vsl], sem_i.at[jbuf]).wait()

    fetch_idx(0, 0)
    fetch_idx(1, 1)
    pltpu.make_async_copy(neg_hbm.at[tslice], table.at[tslice],
                          sem_o.at[0]).wait()

    @pl.when(sid == 0)
    def _():
        pltpu.make_async_copy(neg_hbm.at[rem], table.at[rem],
                              sem_o.at[1]).wait()

    plsc.subcore_barrier()

    def idx_refs(i):
        off = ((i // _IDX_BLK) % 2) * _BLK_E + (i % _IDX_BLK) * _CHUNK
        return (sidx.at[pl.ds(off, _CHUNK)], didx.at[pl.ds(off, _CHUNK)])

    def issue_src(i, b):
        @pl.when(i % _IDX_BLK == 0)
        def _():
            wait_idx(i // _IDX_BLK, (i // _IDX_BLK) % 2)

        s_ix, _ = idx_refs(i)
        pltpu.async_copy(norm_hbm.at[s_ix], rows.at[b], sem_s.at[b])

    def wait_src(i, b):
        s_ix, _ = idx_refs(i)
        pltpu.make_async_copy(norm_hbm.at[s_ix], rows.at[b],
                              sem_s.at[b]).wait()

    def issue_dst_add(i, b):
        _, d_ix = idx_refs(i)
        pltpu.async_copy(table.at[d_ix], rows.at[b], sem_d.at[b], add=True)

    def wait_dst_add(i, b):
        _, d_ix = idx_refs(i)
        pltpu.make_async_copy(table.at[d_ix], rows.at[b],
                              sem_d.at[b]).wait()

    def prefetch_idx(i):
        j2 = i // _IDX_BLK + 2

        @pl.when((i % _IDX_BLK == _IDX_BLK - 1) & (j2 < _N_BLKS))
        def _():
            fetch_idx(j2, j2 % 2)

    def out_slice(i):
        return out_hbm.at[pl.ds(ebase + i * _CHUNK, _CHUNK)]

    def wait_out(i, b):
        pltpu.make_async_copy(rows.at[b], out_slice(i), sem_o.at[b]).wait()

    wait_idx(0, 0)
    for j in range(3):
        s_ixj, _ = idx_refs(j)
        pltpu.async_copy(norm_hbm.at[s_ixj], rows.at[j], sem_s.at[j])

    def body(i0, carry):
        for b2 in range(4):
            i = i0 * 4 + b2
            bg = (b2 + 3) % 4
            if b2 == 0:
                @pl.when(i0 > 0)
                def _():
                    wait_out(i - 1, bg)
            else:
                wait_out(i - 1, bg)
            @pl.when(i + 3 < _CH_PER_W)
            def _():
                issue_src(i + 3, bg)

            wait_src(i, b2)
            issue_dst_add(i, b2)
            wait_dst_add(i, b2)
            prefetch_idx(i)
            pltpu.async_copy(rows.at[b2], out_slice(i), sem_o.at[b2])
        return carry

    lax.fori_loop(0, (_CH_PER_W - 1) // 4, body, 0)

    _L = _CH_PER_W - 1
    wait_out(_L - 1, 3)
    wait_src(_L, 0)
    issue_dst_add(_L, 0)
    wait_dst_add(_L, 0)
    pltpu.async_copy(rows.at[0], out_slice(_L), sem_o.at[0])
    wait_out(_L, 0)


def _norm_body(x_ref, o_ref):
    o_ref[...] = (x_ref[...] - _STAT_MEDIAN) * _STAT_SCALE


def _neg_body(x_ref, n_ref):
    n_ref[...] = (_STAT_MEDIAN - x_ref[...]) * _STAT_SCALE


_shape = jax.ShapeDtypeStruct((_N_NODES, _NODE_FEATS), jnp.float32)
_norm_call = pl.pallas_call(_norm_body, out_shape=_shape)
_neg_call = pl.pallas_call(_neg_body, out_shape=_shape)


def kernel(node_feature, edge_index):
    src = edge_index[0].astype(jnp.int32)
    dst = edge_index[1].astype(jnp.int32)
    norm = _norm_call(node_feature)
    neg = _neg_call(node_feature)
    if _STAT_SCALE == 1.0 and _STAT_MEDIAN == 0.0:
        src_table = node_feature
    else:
        src_table = norm
    edge_feature = _edge_kernel(src_table, neg, src, dst)
    return (norm, edge_feature)
